# Initial kernel scaffold; baseline (speedup 1.0000x reference)
import jax
import jax.numpy as jnp
from jax.experimental import pallas as pl


def kernel(scores, targets, C):
    raise NotImplementedError("write your pallas kernel here")


# speedup vs baseline: 2.3176x; 2.3176x over previous
import jax
import jax.numpy as jnp
from jax.experimental import pallas as pl
from jax.experimental.pallas import tpu as pltpu

B, K = 512, 256
BB = 8
N_IT = 50
EPSM = 1e-8


def _cacis_kernel(s_ref, t_ref, c_ref, o_ref, mt_scr):
    c_all = c_ref[...]
    tot = jnp.sum(c_all, axis=(1, 2), keepdims=True)
    ii = jax.lax.broadcasted_iota(jnp.int32, (1, K, K), 1)
    jj = jax.lax.broadcasted_iota(jnp.int32, (1, K, K), 2)
    diag = jnp.sum(jnp.where(ii == jj, c_all, 0.0), axis=(1, 2), keepdims=True)
    eps_all = jnp.maximum((tot - diag) / float(K * K - K), EPSM)

    f_all = 0.5 * s_ref[...]

    lane = jax.lax.broadcasted_iota(jnp.int32, (1, K), 1)

    alphas, gs, epss, shifts = [], [], [], []
    for b in range(BB):
        c_b = c_all[b]
        f_b = f_all[b:b + 1, :]
        eps_b = eps_all[b]
        a_t = (f_b + c_b).T
        e_t = a_t + f_b
        mmin = jnp.min(e_t, axis=(0, 1), keepdims=True)
        logmt = (mmin - e_t) / eps_b
        mt_b = jnp.exp(logmt)
        mt_scr[b] = mt_b
        g0 = jnp.sum(mt_b, axis=0, keepdims=True) * (2.0 / K)
        alphas.append(jnp.full((1, K), 1.0 / K, dtype=jnp.float32))
        gs.append(g0)
        epss.append(eps_b)
        shifts.append(-mmin / eps_b)

    def fw_body(i, carry):
        gam = 2.0 / (i.astype(jnp.float32) + 2.0)
        out = []
        for b in range(BB):
            al, g = carry[b]
            s_idx = jnp.argmin(g, axis=1)
            s0 = s_idx[0]
            col = mt_scr[b, pl.ds(s0, 1), :]
            al = al * (1.0 - gam) + jnp.where(lane == s0, gam, 0.0)
            g = g * (1.0 - gam) + (2.0 * gam) * col
            out.append((al, g))
        return tuple(out)

    carry = tuple((alphas[b], gs[b]) for b in range(BB))
    carry = jax.lax.fori_loop(0, N_IT, fw_body, carry)

    conjs = []
    for b in range(BB):
        al, g = carry[b]
        val = 0.5 * jnp.sum(al * g, axis=1, keepdims=True)
        conjs.append(-epss[b] * (jnp.log(val) + shifts[b]))
    conj = jnp.concatenate(conjs, axis=0)

    t_all = t_ref[...]
    lane_b = jax.lax.broadcasted_iota(jnp.int32, (BB, K), 1)
    fy = jnp.sum(jnp.where(lane_b == t_all, s_ref[...], 0.0),
                 axis=1, keepdims=True)
    o_ref[...] = conj - fy


def _cacis_call(scores, t2, C, interpret=False):
    return pl.pallas_call(
        _cacis_kernel,
        grid=(B // BB,),
        in_specs=[
            pl.BlockSpec((BB, K), lambda i: (i, 0)),
            pl.BlockSpec((BB, 1), lambda i: (i, 0)),
            pl.BlockSpec((BB, K, K), lambda i: (i, 0, 0)),
        ],
        out_specs=pl.BlockSpec((BB, 1), lambda i: (i, 0)),
        out_shape=jax.ShapeDtypeStruct((B, 1), jnp.float32),
        scratch_shapes=[pltpu.VMEM((BB, K, K), jnp.float32)],
        compiler_params=pltpu.CompilerParams(
            dimension_semantics=("parallel",),
        ),
        name="cacis_loss",
        interpret=interpret,
    )(scores, t2, C)


def kernel(scores, targets, C):
    t2 = targets.astype(jnp.int32).reshape(B, 1)
    per_batch = _cacis_call(scores, t2, C)
    return jnp.mean(per_batch)
